# Initial kernel scaffold; baseline (speedup 1.0000x reference)
import jax
import jax.numpy as jnp
from jax.experimental import pallas as pl


def kernel(feat, label, memory):
    raise NotImplementedError("write your pallas kernel here")


# speedup vs baseline: 3.7471x; 3.7471x over previous
import functools

import jax
import jax.numpy as jnp
from jax.experimental import pallas as pl

NUM_CLS = 1000
CLS_PAD = 1024
FEAT_DIM = 256
BATCH = 16384
ROW_BLK = 512
N_BLK = BATCH // ROW_BLK
EPS = 1e-12
NEG = -1e30


def _seg_kernel(feat_ref, label_ref, sums_ref, cnt_ref):
    i = pl.program_id(0)

    @pl.when(i == 0)
    def _init():
        sums_ref[...] = jnp.zeros_like(sums_ref)
        cnt_ref[...] = jnp.zeros_like(cnt_ref)

    f = feat_ref[...]
    norm = jnp.sqrt(jnp.sum(f * f, axis=1, keepdims=True))
    fn = f / jnp.maximum(norm, EPS)
    lab = label_ref[0, 0, :]
    rows = jax.lax.broadcasted_iota(jnp.int32, (CLS_PAD, ROW_BLK), 0)
    oht = (rows == lab[None, :]).astype(jnp.float32)
    contrib = jax.lax.dot_general(
        oht, fn, (((1,), (0,)), ((), ())),
        preferred_element_type=jnp.float32)
    sums_ref[...] += contrib
    cnt_ref[...] += jnp.sum(oht, axis=1, keepdims=True)


def _center_kernel(sums_ref, cnt_ref, mem_ref, out_ref):
    s = sums_ref[...]
    has = (cnt_ref[...] > 0).astype(jnp.float32)
    n = jnp.sqrt(jnp.sum(s * s, axis=1, keepdims=True))
    bc = (s / jnp.maximum(n, EPS)) * has
    mem = mem_ref[...]
    uw = jnp.sum(mem * bc, axis=1, keepdims=True)
    update_wei = 1.0 - (1.0 - uw) * has
    nm = update_wei * mem + (1.0 - update_wei) * bc
    nn = jnp.sqrt(jnp.sum(nm * nm, axis=1, keepdims=True))
    out_ref[...] = nm / jnp.maximum(nn, EPS)


def _loss_kernel(feat_ref, label_ref, nm_ref, out_ref):
    i = pl.program_id(0)

    @pl.when(i == 0)
    def _init():
        out_ref[...] = jnp.zeros_like(out_ref)

    f = feat_ref[...]
    norm = jnp.sqrt(jnp.sum(f * f, axis=1, keepdims=True))
    fn = f / jnp.maximum(norm, EPS)
    nm = nm_ref[...]
    sims = jax.lax.dot_general(
        fn, nm, (((1,), (1,)), ((), ())),
        preferred_element_type=jnp.float32)
    cols = jax.lax.broadcasted_iota(jnp.int32, (ROW_BLK, CLS_PAD), 1)
    sims = jnp.where(cols < NUM_CLS, sims, NEG)
    m = jnp.max(sims, axis=1, keepdims=True)
    lse = m + jnp.log(jnp.sum(jnp.exp(sims - m), axis=1, keepdims=True))
    lab = label_ref[0, 0, :]
    oh = (lab[:, None] == cols).astype(jnp.float32)
    tgt = jnp.sum(sims * oh, axis=1, keepdims=True)
    part = jnp.sum(lse - tgt) * jnp.float32(1.0 / BATCH)
    out_ref[...] += part


@jax.jit
def kernel(feat, label, memory):
    label3 = label.reshape(N_BLK, 1, ROW_BLK)
    mem_pad = jnp.pad(memory, ((0, CLS_PAD - NUM_CLS), (0, 0)))

    sums, cnt = pl.pallas_call(
        _seg_kernel,
        grid=(N_BLK,),
        in_specs=[
            pl.BlockSpec((ROW_BLK, FEAT_DIM), lambda i: (i, 0)),
            pl.BlockSpec((1, 1, ROW_BLK), lambda i: (i, 0, 0)),
        ],
        out_specs=[
            pl.BlockSpec((CLS_PAD, FEAT_DIM), lambda i: (0, 0)),
            pl.BlockSpec((CLS_PAD, 1), lambda i: (0, 0)),
        ],
        out_shape=[
            jax.ShapeDtypeStruct((CLS_PAD, FEAT_DIM), jnp.float32),
            jax.ShapeDtypeStruct((CLS_PAD, 1), jnp.float32),
        ],
    )(feat, label3)

    new_mem = pl.pallas_call(
        _center_kernel,
        out_shape=jax.ShapeDtypeStruct((CLS_PAD, FEAT_DIM), jnp.float32),
    )(sums, cnt, mem_pad)

    loss = pl.pallas_call(
        _loss_kernel,
        grid=(N_BLK,),
        in_specs=[
            pl.BlockSpec((ROW_BLK, FEAT_DIM), lambda i: (i, 0)),
            pl.BlockSpec((1, 1, ROW_BLK), lambda i: (i, 0, 0)),
            pl.BlockSpec((CLS_PAD, FEAT_DIM), lambda i: (0, 0)),
        ],
        out_specs=pl.BlockSpec((1, 1), lambda i: (0, 0)),
        out_shape=jax.ShapeDtypeStruct((1, 1), jnp.float32),
    )(feat, label3, new_mem)

    return loss[0, 0]


# speedup vs baseline: 4.8245x; 1.2875x over previous
import jax
import jax.numpy as jnp
from jax.experimental import pallas as pl
from jax.experimental.pallas import tpu as pltpu

NUM_CLS = 1000
CLS_PAD = 1024
FEAT_DIM = 256
BATCH = 16384
ROW_BLK = 512
N_BLK = BATCH // ROW_BLK
EPS = 1e-12


def _fused_kernel(feat_ref, label_ref, mem_ref, out_ref,
                  fn_ref, sums_ref, cnt_ref, nm_ref):
    i = pl.program_id(0)

    @pl.when(i == 0)
    def _init():
        sums_ref[...] = jnp.zeros_like(sums_ref)
        cnt_ref[...] = jnp.zeros_like(cnt_ref)
        out_ref[...] = jnp.zeros_like(out_ref)

    @pl.when(i < N_BLK)
    def _accumulate():
        f = feat_ref[...]
        norm = jnp.sqrt(jnp.sum(f * f, axis=1, keepdims=True))
        fn = f / jnp.maximum(norm, EPS)
        fn_ref[pl.ds(i * ROW_BLK, ROW_BLK), :] = fn
        lab = label_ref[0, 0, :]
        rows = jax.lax.broadcasted_iota(jnp.int32, (CLS_PAD, ROW_BLK), 0)
        oht = (rows == lab[None, :]).astype(jnp.float32)
        sums_ref[...] += jax.lax.dot_general(
            oht, fn, (((1,), (0,)), ((), ())),
            preferred_element_type=jnp.float32)
        cnt_ref[...] += jnp.sum(oht, axis=1, keepdims=True)

    @pl.when(i == N_BLK)
    def _center():
        s = sums_ref[...]
        has = (cnt_ref[...] > 0).astype(jnp.float32)
        n = jnp.sqrt(jnp.sum(s * s, axis=1, keepdims=True))
        bc = (s / jnp.maximum(n, EPS)) * has
        mem = mem_ref[...]
        uw = jnp.sum(mem * bc, axis=1, keepdims=True)
        update_wei = 1.0 - (1.0 - uw) * has
        nm = update_wei * mem + (1.0 - update_wei) * bc
        nn = jnp.sqrt(jnp.sum(nm * nm, axis=1, keepdims=True))
        nm = nm / jnp.maximum(nn, EPS)
        nm_ref[...] = nm
        out_ref[...] -= jnp.sum(s * nm) * (1.0 / BATCH)

    @pl.when(i >= N_BLK)
    def _loss():
        j = i - N_BLK
        fn = fn_ref[pl.ds(j * ROW_BLK, ROW_BLK), :]
        sims = jax.lax.dot_general(
            fn, nm_ref[...], (((1,), (1,)), ((), ())),
            preferred_element_type=jnp.float32)
        se = jnp.sum(jnp.exp(sims), axis=1, keepdims=True)
        lse = jnp.log(se - float(CLS_PAD - NUM_CLS))
        out_ref[...] += jnp.sum(lse) * (1.0 / BATCH)


@jax.jit
def kernel(feat, label, memory):
    label3 = label.reshape(N_BLK, 1, ROW_BLK)
    mem_pad = jnp.pad(memory, ((0, CLS_PAD - NUM_CLS), (0, 0)))

    loss = pl.pallas_call(
        _fused_kernel,
        grid=(2 * N_BLK,),
        in_specs=[
            pl.BlockSpec((ROW_BLK, FEAT_DIM),
                         lambda i: (jnp.minimum(i, N_BLK - 1), 0)),
            pl.BlockSpec((1, 1, ROW_BLK),
                         lambda i: (jnp.minimum(i, N_BLK - 1), 0, 0)),
            pl.BlockSpec((CLS_PAD, FEAT_DIM), lambda i: (0, 0)),
        ],
        out_specs=pl.BlockSpec((1, 1), lambda i: (0, 0)),
        out_shape=jax.ShapeDtypeStruct((1, 1), jnp.float32),
        scratch_shapes=[
            pltpu.VMEM((BATCH, FEAT_DIM), jnp.float32),
            pltpu.VMEM((CLS_PAD, FEAT_DIM), jnp.float32),
            pltpu.VMEM((CLS_PAD, 1), jnp.float32),
            pltpu.VMEM((CLS_PAD, FEAT_DIM), jnp.float32),
        ],
    )(feat, label3, mem_pad)

    return loss[0, 0]


# speedup vs baseline: 4.8547x; 1.0063x over previous
import jax
import jax.numpy as jnp
from jax import lax
from jax.experimental import pallas as pl
from jax.experimental.pallas import tpu as pltpu

NUM_CLS = 1000
CLS_PAD = 1024
FEAT_DIM = 256
BATCH = 16384
ROW_BLK = 512
N_BLK = BATCH // ROW_BLK
EPS = 1e-12


def _normalize(f):
    r = lax.rsqrt(jnp.sum(f * f, axis=1, keepdims=True))
    return f * jnp.minimum(r, 1.0 / EPS)


def _fused_kernel(feat_ref, label_ref, mem_ref, out_ref,
                  fn_ref, sums_ref, cnt_ref, nm_ref):
    i = pl.program_id(0)

    @pl.when(i == 0)
    def _init():
        sums_ref[...] = jnp.zeros_like(sums_ref)
        cnt_ref[...] = jnp.zeros_like(cnt_ref)
        out_ref[...] = jnp.zeros_like(out_ref)

    @pl.when(i < N_BLK)
    def _accumulate():
        fn = _normalize(feat_ref[...]).astype(jnp.bfloat16)
        fn_ref[pl.ds(i * ROW_BLK, ROW_BLK), :] = fn
        lab = label_ref[0, 0, :]
        rows = lax.broadcasted_iota(jnp.int32, (CLS_PAD, ROW_BLK), 0)
        oht = (rows == lab[None, :]).astype(jnp.bfloat16)
        sums_ref[...] += lax.dot_general(
            oht, fn, (((1,), (0,)), ((), ())),
            preferred_element_type=jnp.float32)
        cnt_ref[...] += jnp.sum(oht.astype(jnp.float32), axis=1, keepdims=True)

    @pl.when(i == N_BLK)
    def _center():
        s = sums_ref[...]
        has = (cnt_ref[...] > 0).astype(jnp.float32)
        bc = _normalize(s) * has
        mem = mem_ref[...]
        uw = jnp.sum(mem * bc, axis=1, keepdims=True)
        update_wei = 1.0 - (1.0 - uw) * has
        nm = update_wei * mem + (1.0 - update_wei) * bc
        nm = _normalize(nm)
        nm_ref[...] = nm.astype(jnp.bfloat16)
        out_ref[...] -= jnp.sum(s * nm) * (1.0 / BATCH)

    @pl.when(i >= N_BLK)
    def _loss():
        j = i - N_BLK
        fn = fn_ref[pl.ds(j * ROW_BLK, ROW_BLK), :]
        sims = lax.dot_general(
            fn, nm_ref[...], (((1,), (1,)), ((), ())),
            preferred_element_type=jnp.float32)
        se = jnp.sum(jnp.exp(sims), axis=1, keepdims=True)
        lse = jnp.log(se - float(CLS_PAD - NUM_CLS))
        out_ref[...] += jnp.sum(lse) * (1.0 / BATCH)


@jax.jit
def kernel(feat, label, memory):
    label3 = label.reshape(N_BLK, 1, ROW_BLK)
    mem_pad = jnp.pad(memory, ((0, CLS_PAD - NUM_CLS), (0, 0)))

    loss = pl.pallas_call(
        _fused_kernel,
        grid=(2 * N_BLK,),
        in_specs=[
            pl.BlockSpec((ROW_BLK, FEAT_DIM),
                         lambda i: (jnp.minimum(i, N_BLK - 1), 0)),
            pl.BlockSpec((1, 1, ROW_BLK),
                         lambda i: (jnp.minimum(i, N_BLK - 1), 0, 0)),
            pl.BlockSpec((CLS_PAD, FEAT_DIM), lambda i: (0, 0)),
        ],
        out_specs=pl.BlockSpec((1, 1), lambda i: (0, 0)),
        out_shape=jax.ShapeDtypeStruct((1, 1), jnp.float32),
        scratch_shapes=[
            pltpu.VMEM((BATCH, FEAT_DIM), jnp.bfloat16),
            pltpu.VMEM((CLS_PAD, FEAT_DIM), jnp.float32),
            pltpu.VMEM((CLS_PAD, 1), jnp.float32),
            pltpu.VMEM((CLS_PAD, FEAT_DIM), jnp.bfloat16),
        ],
    )(feat, label3, mem_pad)

    return loss[0, 0]
